# Initial kernel scaffold; baseline (speedup 1.0000x reference)
import jax
import jax.numpy as jnp
from jax.experimental import pallas as pl


def kernel(x, embedding):
    raise NotImplementedError("write your pallas kernel here")


# speedup vs baseline: 6.0121x; 6.0121x over previous
import functools

import jax
import jax.numpy as jnp
from jax import lax
from jax.experimental import pallas as pl
from jax.experimental.pallas import tpu as pltpu
from jax.experimental.pallas import tpu_sc as plsc

NC = 2
NS = 16
NW = NC * NS
D = 64

SUB = 128
SUBS_PER_CHUNK = 4
CH = SUB * SUBS_PER_CHUNK


def _gather_kernel(B: int):
    b_per_w = B // NW
    n_chunks = b_per_w // CH
    mesh = plsc.VectorSubcoreMesh(core_axis_name="c", subcore_axis_name="s")

    @functools.partial(
        pl.kernel,
        mesh=mesh,
        out_type=jax.ShapeDtypeStruct((B, D), jnp.float32),
        scratch_types=[
            pltpu.VMEM((CH,), jnp.int32),
            pltpu.VMEM((CH, D), jnp.float32),
            pltpu.SemaphoreType.DMA,
        ],
        compiler_params=pltpu.CompilerParams(use_tc_tiling_on_sc=False),
    )
    def k(idx_hbm, table_hbm, out_hbm, idx_v, rows_v, sem):
        wid = lax.axis_index("s") * NC + lax.axis_index("c")
        base = wid * b_per_w

        def body(i, carry):
            off = base + i * CH
            pltpu.sync_copy(idx_hbm.at[pl.ds(off, CH)], idx_v)
            copies = [
                pltpu.async_copy(
                    table_hbm.at[idx_v.at[pl.ds(j * SUB, SUB)]],
                    rows_v.at[pl.ds(j * SUB, SUB)],
                    sem,
                )
                for j in range(SUBS_PER_CHUNK)
            ]
            for c in copies:
                c.wait()
            pltpu.sync_copy(rows_v, out_hbm.at[pl.ds(off, CH)])
            return carry

        lax.fori_loop(0, n_chunks, body, 0)

    return k


def kernel(x, embedding):
    n, m = x.shape
    B = n * m
    idx = x.reshape(B).astype(jnp.int32)
    out = _gather_kernel(B)(idx, embedding)
    return out.reshape(n, m, D)


# speedup vs baseline: 6.5199x; 1.0845x over previous
import functools

import jax
import jax.numpy as jnp
from jax import lax
from jax.experimental import pallas as pl
from jax.experimental.pallas import tpu as pltpu
from jax.experimental.pallas import tpu_sc as plsc

NC = 2
NS = 16
NW = NC * NS
D = 64

SUB = 128
SUBS_PER_CHUNK = 4
CH = SUB * SUBS_PER_CHUNK


def _gather_kernel(B: int):
    b_per_w = B // NW
    n_chunks = b_per_w // CH
    assert n_chunks % 2 == 0
    K = n_chunks // 2
    mesh = plsc.VectorSubcoreMesh(core_axis_name="c", subcore_axis_name="s")

    @functools.partial(
        pl.kernel,
        mesh=mesh,
        out_type=jax.ShapeDtypeStruct((B, D), jnp.float32),
        scratch_types=[
            pltpu.VMEM((CH,), jnp.int32),
            pltpu.VMEM((CH,), jnp.int32),
            pltpu.VMEM((CH, D), jnp.float32),
            pltpu.VMEM((CH, D), jnp.float32),
            pltpu.SemaphoreType.DMA,
            pltpu.SemaphoreType.DMA,
            pltpu.SemaphoreType.DMA,
            pltpu.SemaphoreType.DMA,
        ],
        compiler_params=pltpu.CompilerParams(use_tc_tiling_on_sc=False),
    )
    def k(idx_hbm, table_hbm, out_hbm, idx0, idx1, rows0, rows1,
          gsem0, gsem1, wsem0, wsem1):
        wid = lax.axis_index("s") * NC + lax.axis_index("c")
        base = wid * b_per_w

        def fire_gathers(idx_v, rows_v, sem):
            for j in range(SUBS_PER_CHUNK):
                pltpu.async_copy(
                    table_hbm.at[idx_v.at[pl.ds(j * SUB, SUB)]],
                    rows_v.at[pl.ds(j * SUB, SUB)],
                    sem,
                )

        def drain_gathers(idx_v, rows_v, sem):
            for j in range(SUBS_PER_CHUNK):
                pltpu.make_async_copy(
                    table_hbm.at[idx_v.at[pl.ds(j * SUB, SUB)]],
                    rows_v.at[pl.ds(j * SUB, SUB)],
                    sem,
                ).wait()

        def drain_write(rows_v, sem):
            pltpu.make_async_copy(rows_v, out_hbm.at[pl.ds(0, CH)], sem).wait()

        pltpu.sync_copy(idx_hbm.at[pl.ds(base, CH)], idx0)
        fire_gathers(idx0, rows0, gsem0)

        def body(kk, carry):
            o0 = base + (2 * kk) * CH
            o1 = o0 + CH
            o2 = o0 + 2 * CH
            pltpu.sync_copy(idx_hbm.at[pl.ds(o1, CH)], idx1)

            @pl.when(kk > 0)
            def _():
                drain_write(rows1, wsem1)

            fire_gathers(idx1, rows1, gsem1)
            drain_gathers(idx0, rows0, gsem0)
            pltpu.async_copy(rows0, out_hbm.at[pl.ds(o0, CH)], wsem0)

            @pl.when(kk < K - 1)
            def _():
                pltpu.sync_copy(idx_hbm.at[pl.ds(o2, CH)], idx0)
                drain_write(rows0, wsem0)
                fire_gathers(idx0, rows0, gsem0)

            drain_gathers(idx1, rows1, gsem1)
            pltpu.async_copy(rows1, out_hbm.at[pl.ds(o1, CH)], wsem1)
            return carry

        lax.fori_loop(0, K, body, 0)
        drain_write(rows0, wsem0)
        drain_write(rows1, wsem1)

    return k


def kernel(x, embedding):
    n, m = x.shape
    B = n * m
    idx = x.reshape(B).astype(jnp.int32)
    out = _gather_kernel(B)(idx, embedding)
    return out.reshape(n, m, D)
